# baseline (speedup 1.0000x reference)
import functools

import jax
import jax.numpy as jnp
from jax import lax
from jax.experimental import pallas as pl
from jax.experimental.pallas import tpu as pltpu
from jax.experimental.pallas import tpu_sc as plsc

_NUM_GROUPS = 64
_NUM_CLASSES = 80
_MH = 64
_MW = 64
_NUM_PTS = 2048
_NUM_SAMPLED = 6144
_K = 1536
_NUM_RAND = 512
_NVEC = _NUM_SAMPLED // 16
_KVEC = _K // 16
_RPT = 2

_mesh = plsc.VectorSubcoreMesh(core_axis_name="c", subcore_axis_name="s")


@functools.partial(
    pl.kernel,
    out_type=[
        jax.ShapeDtypeStruct((_NUM_GROUPS * _K,), jnp.float32),
        jax.ShapeDtypeStruct((_NUM_GROUPS * _K,), jnp.float32),
    ],
    mesh=_mesh,
    scratch_types=[
        pltpu.VMEM((_RPT, _MH * _MW), jnp.float32),
        pltpu.VMEM((8,), jnp.int32),
        pltpu.VMEM((_NUM_SAMPLED,), jnp.float32),
        pltpu.VMEM((_NUM_SAMPLED,), jnp.float32),
        pltpu.VMEM((_NUM_SAMPLED,), jnp.int32),
        pltpu.VMEM((_NUM_SAMPLED,), jnp.int32),
        pltpu.VMEM((_NUM_SAMPLED,), jnp.int32),
        pltpu.VMEM((_NUM_SAMPLED,), jnp.int32),
        pltpu.VMEM((_NUM_SAMPLED,), jnp.int32),
        pltpu.VMEM((256,), jnp.int32),
        pltpu.VMEM((256,), jnp.int32),
        pltpu.VMEM((_K,), jnp.float32),
        pltpu.VMEM((_K,), jnp.float32),
        pltpu.SemaphoreType.DMA,
    ],
    compiler_params=pltpu.CompilerParams(needs_layout_passes=False),
)
def _sc_kernel(map_hbm, rowidx_hbm, xs_hbm, ys_hbm, selx_hbm, sely_hbm,
               map_v, rowidx_v, xs_v, ys_v, ka_v, kb_v, ia_v, ib_v, prk_v,
               hist_v, cnt_v, outx_v, outy_v, dma_sem):
    wid = lax.axis_index("s") * 2 + lax.axis_index("c")
    lanes = lax.iota(jnp.int32, 16)
    zeros16 = jnp.zeros((16,), jnp.int32)

    pltpu.sync_copy(rowidx_hbm.at[pl.ds(wid * 8, 8)], rowidx_v)
    pltpu.async_copy(map_hbm.at[rowidx_v.at[pl.ds(0, _RPT)]], map_v,
                     dma_sem).wait()

    for t in range(_RPT):
        roi = wid * _RPT + t
        pltpu.sync_copy(xs_hbm.at[pl.ds(roi * _NUM_SAMPLED, _NUM_SAMPLED)],
                        xs_v)
        pltpu.sync_copy(ys_hbm.at[pl.ds(roi * _NUM_SAMPLED, _NUM_SAMPLED)],
                        ys_v)
        tvec = zeros16 + t

        @plsc.parallel_loop(0, _NVEC, unroll=4)
        def sample_body(i):
            sl = pl.ds(i * 16, 16)
            cx = xs_v[sl]
            cy = ys_v[sl]
            gx = 2.0 * cx - 1.0
            gy = 2.0 * cy - 1.0
            x = ((gx + 1.0) * float(_MW) - 1.0) / 2.0
            y = ((gy + 1.0) * float(_MH) - 1.0) / 2.0
            xi = x.astype(jnp.int32)
            yi = y.astype(jnp.int32)
            x0f = jnp.where(x < 0.0, -1.0, xi.astype(jnp.float32))
            y0f = jnp.where(y < 0.0, -1.0, yi.astype(jnp.float32))
            wx1 = x - x0f
            wx0 = 1.0 - wx1
            wy1 = y - y0f
            wy0 = 1.0 - wy1
            x1f = x0f + 1.0
            y1f = y0f + 1.0
            fvx0 = jnp.where(x0f >= 0.0, 1.0, 0.0)
            fvx1 = jnp.where(x1f <= float(_MW - 1), 1.0, 0.0)
            fvy0 = jnp.where(y0f >= 0.0, 1.0, 0.0)
            fvy1 = jnp.where(y1f <= float(_MH - 1), 1.0, 0.0)
            ix0 = jnp.maximum(x0f.astype(jnp.int32), 0)
            iy0 = jnp.maximum(y0f.astype(jnp.int32), 0)
            ix1 = jnp.minimum(x1f.astype(jnp.int32), _MW - 1)
            iy1 = jnp.minimum(y1f.astype(jnp.int32), _MH - 1)
            row0 = iy0 * _MW
            row1 = iy1 * _MW
            g00 = plsc.load_gather(map_v, [tvec, row0 + ix0]) * (fvx0 * fvy0)
            g10 = plsc.load_gather(map_v, [tvec, row0 + ix1]) * (fvx1 * fvy0)
            g01 = plsc.load_gather(map_v, [tvec, row1 + ix0]) * (fvx0 * fvy1)
            g11 = plsc.load_gather(map_v, [tvec, row1 + ix1]) * (fvx1 * fvy1)
            r = (g00 * (wx0 * wy0) + g10 * (wx1 * wy0)
                 + g01 * (wx0 * wy1) + g11 * (wx1 * wy1))
            u = -jnp.abs(r)
            ka_v[sl] = plsc.bitcast(u, jnp.int32)

        for p in range(4):
            shift = 8 * p
            src_k, dst_k = (ka_v, kb_v) if p % 2 == 0 else (kb_v, ka_v)
            src_i, dst_i = (ia_v, ib_v) if p % 2 == 0 else (ib_v, ia_v)

            def zero_body(i, _):
                hist_v[pl.ds(i * 16, 16)] = zeros16
                return 0

            lax.fori_loop(0, 16, zero_body, 0)

            def hist_body(i, _):
                sl = pl.ds(i * 16, 16)
                k = src_k[sl]
                d = lax.shift_right_logical(k, shift) & 255
                rank1, mlast = plsc.scan_count(d)
                base = plsc.load_gather(hist_v, [d])
                prk_v[sl] = base + rank1 - 1
                plsc.store_scatter(hist_v, [d], base + rank1, mask=mlast)
                return 0

            lax.fori_loop(0, _NVEC, hist_body, 0, unroll=8)

            def scan_body(c, carry):
                sl = pl.ds(c * 16, 16)
                v = hist_v[sl]
                inc = plsc.cumsum(v)
                cnt_v[sl] = inc - v + carry
                return carry + jnp.max(inc)

            lax.fori_loop(0, 16, scan_body, jnp.int32(0))

            @plsc.parallel_loop(0, _NVEC, unroll=4)
            def perm_body(i):
                sl = pl.ds(i * 16, 16)
                k = src_k[sl]
                d = lax.shift_right_logical(k, shift) & 255
                base = plsc.load_gather(cnt_v, [d])
                pos = base + prk_v[sl]
                plsc.store_scatter(dst_k, [pos], k)
                if p == 0:
                    idxval = i * 16 + lanes
                else:
                    idxval = src_i[sl]
                plsc.store_scatter(dst_i, [pos], idxval)

        @plsc.parallel_loop(0, _KVEC, unroll=4)
        def out_body(j):
            sl = pl.ds(j * 16, 16)
            sidx = ia_v[sl]
            outx_v[sl] = plsc.load_gather(xs_v, [sidx])
            outy_v[sl] = plsc.load_gather(ys_v, [sidx])

        pltpu.sync_copy(outx_v, selx_hbm.at[pl.ds(roi * _K, _K)])
        pltpu.sync_copy(outy_v, sely_hbm.at[pl.ds(roi * _K, _K)])


def kernel(in_map, labels):
    in_map = in_map.astype(jnp.float32)
    labels = labels.astype(jnp.int32)
    k1, k2 = jax.random.split(jax.random.key(42))
    point_coords = jax.random.uniform(
        k1, (_NUM_GROUPS, _NUM_SAMPLED, 2), dtype=jnp.float32)
    rand_coords = jax.random.uniform(
        k2, (_NUM_GROUPS, _NUM_RAND, 2), dtype=jnp.float32)

    map2d = in_map.reshape(_NUM_GROUPS * _NUM_CLASSES, _MH * _MW)
    row_idx = jnp.arange(_NUM_GROUPS, dtype=jnp.int32) * _NUM_CLASSES + labels
    row_idx = jnp.zeros((_NUM_GROUPS // _RPT, 8), jnp.int32).at[
        :, :_RPT].set(row_idx.reshape(-1, _RPT)).reshape(-1)

    xs = point_coords[..., 0].reshape(-1)
    ys = point_coords[..., 1].reshape(-1)
    selx, sely = _sc_kernel(map2d, row_idx, xs, ys)
    sel = jnp.stack(
        [selx.reshape(_NUM_GROUPS, _K), sely.reshape(_NUM_GROUPS, _K)],
        axis=-1)
    return jnp.concatenate([sel, rand_coords], axis=1)


# speedup vs baseline: 1.0031x; 1.0031x over previous
import functools

import jax
import jax.numpy as jnp
from jax import lax
from jax.experimental import pallas as pl
from jax.experimental.pallas import tpu as pltpu
from jax.experimental.pallas import tpu_sc as plsc

_NUM_GROUPS = 64
_NUM_CLASSES = 80
_MH = 64
_MW = 64
_NUM_PTS = 2048
_NUM_SAMPLED = 6144
_K = 1536
_NUM_RAND = 512
_NVEC = _NUM_SAMPLED // 16
_KVEC = _K // 16
_RPT = 2

_mesh = plsc.VectorSubcoreMesh(core_axis_name="c", subcore_axis_name="s")


@functools.partial(
    pl.kernel,
    out_type=[
        jax.ShapeDtypeStruct((_NUM_GROUPS * _K,), jnp.float32),
        jax.ShapeDtypeStruct((_NUM_GROUPS * _K,), jnp.float32),
    ],
    mesh=_mesh,
    scratch_types=[
        pltpu.VMEM((_RPT, _MH * _MW), jnp.float32),
        pltpu.VMEM((8,), jnp.int32),
        pltpu.VMEM((_NUM_SAMPLED,), jnp.float32),
        pltpu.VMEM((_NUM_SAMPLED,), jnp.float32),
        pltpu.VMEM((_NUM_SAMPLED,), jnp.int32),
        pltpu.VMEM((_NUM_SAMPLED,), jnp.int32),
        pltpu.VMEM((_NUM_SAMPLED,), jnp.int32),
        pltpu.VMEM((_NUM_SAMPLED,), jnp.int32),
        pltpu.VMEM((_NUM_SAMPLED,), jnp.int32),
        pltpu.VMEM((256,), jnp.int32),
        pltpu.VMEM((256,), jnp.int32),
        pltpu.VMEM((_K,), jnp.float32),
        pltpu.VMEM((_K,), jnp.float32),
        pltpu.SemaphoreType.DMA,
    ],
    compiler_params=pltpu.CompilerParams(needs_layout_passes=False),
)
def _sc_kernel(map_hbm, rowidx_hbm, xs_hbm, ys_hbm, selx_hbm, sely_hbm,
               map_v, rowidx_v, xs_v, ys_v, ka_v, kb_v, ia_v, ib_v, prk_v,
               hist_v, cnt_v, outx_v, outy_v, dma_sem):
    wid = lax.axis_index("s") * 2 + lax.axis_index("c")
    lanes = lax.iota(jnp.int32, 16)
    zeros16 = jnp.zeros((16,), jnp.int32)

    pltpu.sync_copy(rowidx_hbm.at[pl.ds(wid * 8, 8)], rowidx_v)
    pltpu.async_copy(map_hbm.at[rowidx_v.at[pl.ds(0, _RPT)]], map_v,
                     dma_sem).wait()

    for t in range(_RPT):
        roi = wid * _RPT + t
        pltpu.sync_copy(xs_hbm.at[pl.ds(roi * _NUM_SAMPLED, _NUM_SAMPLED)],
                        xs_v)
        pltpu.sync_copy(ys_hbm.at[pl.ds(roi * _NUM_SAMPLED, _NUM_SAMPLED)],
                        ys_v)
        tvec = zeros16 + t

        @plsc.parallel_loop(0, _NVEC, unroll=4)
        def sample_body(i):
            sl = pl.ds(i * 16, 16)
            cx = xs_v[sl]
            cy = ys_v[sl]
            gx = 2.0 * cx - 1.0
            gy = 2.0 * cy - 1.0
            x = ((gx + 1.0) * float(_MW) - 1.0) / 2.0
            y = ((gy + 1.0) * float(_MH) - 1.0) / 2.0
            xi = x.astype(jnp.int32)
            yi = y.astype(jnp.int32)
            x0f = jnp.where(x < 0.0, -1.0, xi.astype(jnp.float32))
            y0f = jnp.where(y < 0.0, -1.0, yi.astype(jnp.float32))
            wx1 = x - x0f
            wx0 = 1.0 - wx1
            wy1 = y - y0f
            wy0 = 1.0 - wy1
            x1f = x0f + 1.0
            y1f = y0f + 1.0
            fvx0 = jnp.where(x0f >= 0.0, 1.0, 0.0)
            fvx1 = jnp.where(x1f <= float(_MW - 1), 1.0, 0.0)
            fvy0 = jnp.where(y0f >= 0.0, 1.0, 0.0)
            fvy1 = jnp.where(y1f <= float(_MH - 1), 1.0, 0.0)
            ix0 = jnp.maximum(x0f.astype(jnp.int32), 0)
            iy0 = jnp.maximum(y0f.astype(jnp.int32), 0)
            ix1 = jnp.minimum(x1f.astype(jnp.int32), _MW - 1)
            iy1 = jnp.minimum(y1f.astype(jnp.int32), _MH - 1)
            row0 = iy0 * _MW
            row1 = iy1 * _MW
            g00 = plsc.load_gather(map_v, [tvec, row0 + ix0]) * (fvx0 * fvy0)
            g10 = plsc.load_gather(map_v, [tvec, row0 + ix1]) * (fvx1 * fvy0)
            g01 = plsc.load_gather(map_v, [tvec, row1 + ix0]) * (fvx0 * fvy1)
            g11 = plsc.load_gather(map_v, [tvec, row1 + ix1]) * (fvx1 * fvy1)
            r = (g00 * (wx0 * wy0) + g10 * (wx1 * wy0)
                 + g01 * (wx0 * wy1) + g11 * (wx1 * wy1))
            u = -jnp.abs(r)
            ka_v[sl] = plsc.bitcast(u, jnp.int32)

        for p in range(4):
            shift = 8 * p
            src_k, dst_k = (ka_v, kb_v) if p % 2 == 0 else (kb_v, ka_v)
            src_i, dst_i = (ia_v, ib_v) if p % 2 == 0 else (ib_v, ia_v)

            def zero_body(i, _):
                hist_v[pl.ds(i * 16, 16)] = zeros16
                return 0

            lax.fori_loop(0, 16, zero_body, 0)

            def hist_body(i, _):
                sl = pl.ds(i * 16, 16)
                k = src_k[sl]
                d = lax.shift_right_logical(k, shift) & 255
                rank1, mlast = plsc.scan_count(d)
                base = plsc.load_gather(hist_v, [d])
                prk_v[sl] = base + rank1 - 1
                plsc.store_scatter(hist_v, [d], base + rank1, mask=mlast)
                return 0

            lax.fori_loop(0, _NVEC, hist_body, 0, unroll=4)

            def scan_body(c, carry):
                sl = pl.ds(c * 16, 16)
                v = hist_v[sl]
                inc = plsc.cumsum(v)
                cnt_v[sl] = inc - v + carry
                return carry + jnp.max(inc)

            lax.fori_loop(0, 16, scan_body, jnp.int32(0))

            @plsc.parallel_loop(0, _NVEC, unroll=4)
            def perm_body(i):
                sl = pl.ds(i * 16, 16)
                k = src_k[sl]
                d = lax.shift_right_logical(k, shift) & 255
                base = plsc.load_gather(cnt_v, [d])
                pos = base + prk_v[sl]
                plsc.store_scatter(dst_k, [pos], k)
                if p == 0:
                    idxval = i * 16 + lanes
                else:
                    idxval = src_i[sl]
                plsc.store_scatter(dst_i, [pos], idxval)

        @plsc.parallel_loop(0, _KVEC, unroll=4)
        def out_body(j):
            sl = pl.ds(j * 16, 16)
            sidx = ia_v[sl]
            outx_v[sl] = plsc.load_gather(xs_v, [sidx])
            outy_v[sl] = plsc.load_gather(ys_v, [sidx])

        pltpu.sync_copy(outx_v, selx_hbm.at[pl.ds(roi * _K, _K)])
        pltpu.sync_copy(outy_v, sely_hbm.at[pl.ds(roi * _K, _K)])


def kernel(in_map, labels):
    in_map = in_map.astype(jnp.float32)
    labels = labels.astype(jnp.int32)
    k1, k2 = jax.random.split(jax.random.key(42))
    point_coords = jax.random.uniform(
        k1, (_NUM_GROUPS, _NUM_SAMPLED, 2), dtype=jnp.float32)
    rand_coords = jax.random.uniform(
        k2, (_NUM_GROUPS, _NUM_RAND, 2), dtype=jnp.float32)

    map2d = in_map.reshape(_NUM_GROUPS * _NUM_CLASSES, _MH * _MW)
    row_idx = jnp.arange(_NUM_GROUPS, dtype=jnp.int32) * _NUM_CLASSES + labels
    row_idx = jnp.zeros((_NUM_GROUPS // _RPT, 8), jnp.int32).at[
        :, :_RPT].set(row_idx.reshape(-1, _RPT)).reshape(-1)

    xs = point_coords[..., 0].reshape(-1)
    ys = point_coords[..., 1].reshape(-1)
    selx, sely = _sc_kernel(map2d, row_idx, xs, ys)
    sel = jnp.stack(
        [selx.reshape(_NUM_GROUPS, _K), sely.reshape(_NUM_GROUPS, _K)],
        axis=-1)
    return jnp.concatenate([sel, rand_coords], axis=1)
